# baseline (speedup 1.0000x reference)
import functools

import jax
import jax.numpy as jnp
from jax.experimental import pallas as pl
from jax.experimental.pallas import tpu as pltpu
from jax.experimental.pallas import tpu_sc as plsc

B = 16384
D = 32
K = 5
GROUP = 128
NG = B // GROUP
RBLK = 256
CBLK = 4096
SEL_RBLK = 512
NEG = -3.0e38


def _prep_body(x_ref, xb_ref, xt_ref, n_ref, nt_ref):
    x = x_ref[...]
    xb_ref[...] = x.astype(jnp.bfloat16)
    xt = x.T
    xt_ref[...] = xt.astype(jnp.bfloat16)
    n_ref[...] = jnp.sqrt(jnp.sum(x * x + 1e-6, axis=1, keepdims=True))
    nt_ref[...] = jnp.sqrt(jnp.sum(xt * xt + 1e-6, axis=0, keepdims=True))


def _sim_body(xkb_ref, xtq_ref, nk_ref, ntq_ref, s_ref, l_ref, sidx_ref,
              lsc, gsc):
    r = pl.program_id(0)
    c = pl.program_id(1)
    nc = pl.num_programs(1)
    gpc = CBLK // GROUP

    fz = jax.lax.dot_general(
        xkb_ref[...], xtq_ref[...],
        (((1,), (0,)), ((), ())),
        preferred_element_type=jnp.float32,
    )
    s = (fz * (1.0 / nk_ref[...])) * (1.0 / ntq_ref[...])
    s_ref[...] = s.T.reshape(RBLK, gpc, GROUP)

    g = jnp.max(s.reshape(gpc, GROUP, RBLK), axis=1)
    gsc[c] = g

    @pl.when(c == 0)
    def _():
        lsc[...] = jnp.zeros((1, RBLK), jnp.float32)

    lsc[...] += jnp.sum(jnp.exp(s), axis=0, keepdims=True)

    @pl.when(c == nc - 1)
    def _():
        l_ref[...] = lsc[...]
        cur = gsc[...]
        gid = (jax.lax.broadcasted_iota(jnp.int32, (nc, gpc, RBLK), 0) * gpc
               + jax.lax.broadcasted_iota(jnp.int32, (nc, gpc, RBLK), 1))
        picks = []
        for _j in range(K):
            mv = jnp.max(jnp.max(cur, axis=0), axis=0,
                         keepdims=True)
            am = jnp.min(jnp.min(jnp.where(cur >= mv, gid, NG), axis=0),
                         axis=0, keepdims=True)
            picks.append(am)
            cur = jnp.where(gid == am, NEG, cur)
        g5 = jnp.concatenate(picks, axis=0).T
        rows = r * RBLK + jax.lax.broadcasted_iota(jnp.int32, (RBLK, K), 0)
        sidx_ref[...] = rows * NG + g5


def _select_body(cand_ref, sidx_ref, l_ref, w_ref, ei_ref):
    r = pl.program_id(0)
    w = K * GROUP
    x = cand_ref[...]
    lane = jax.lax.broadcasted_iota(jnp.int32, (SEL_RBLK, w), 1)
    rows = r * SEL_RBLK + jax.lax.broadcasted_iota(
        jnp.int32, (SEL_RBLK, K), 0)
    g5 = sidx_ref[...] - rows * NG

    vals = []
    eidx = []
    for _t in range(K):
        mv = jnp.max(x, axis=-1, keepdims=True)
        am = jnp.min(jnp.where(x >= mv, lane, w), axis=-1,
                     keepdims=True)
        slot = am // GROUP
        off = am - slot * GROUP
        gsel = jnp.zeros_like(slot)
        for j in range(K):
            gsel = jnp.where(slot == j, g5[:, j:j + 1], gsel)
        eidx.append(gsel * GROUP + off)
        vals.append(mv)
        x = jnp.where(lane == am, NEG, x)

    v = jnp.concatenate(vals, axis=1)
    lcol = l_ref[...].T
    p = jnp.exp(v) / lcol
    pm = jnp.max(p, axis=-1, keepdims=True)
    e = jnp.exp(p - pm)
    w_ref[...] = e / jnp.sum(e, axis=-1, keepdims=True)
    ei_ref[...] = jnp.concatenate(eidx, axis=1)


def _combine_body(g_ref, w_ref, o_ref):
    g = g_ref[...]
    w = w_ref[...]
    acc = w[:, 0:1] * g[:, 0, 0:D]
    for j in range(1, K):
        acc = acc + w[:, j:j + 1] * g[:, j, 0:D]
    o_ref[...] = acc


def _sc_gather(table, idx, value_dim, window):
    n = idx.shape[1]
    rows = table.size // value_dim
    mesh = plsc.VectorSubcoreMesh(core_axis_name="core",
                                  subcore_axis_name="subcore")

    @pl.kernel(out_type=jax.ShapeDtypeStruct((n, value_dim), table.dtype),
               mesh=mesh)
    def kern(x_hbm, i_hbm, o_hbm):
        x2d = x_hbm.reshape(rows, value_dim)

        def body(i_vmem, o_vmem):
            pltpu.sync_copy(x2d.at[i_vmem.at[0]], o_vmem)

        pltpu.emit_pipeline(
            body,
            grid=(n // window,),
            in_specs=[pl.BlockSpec((1, window), index_map=lambda i: (0, i))],
            out_specs=[pl.BlockSpec((window, value_dim),
                                    index_map=lambda i: (i, 0))],
            core_axis_name=("core", "subcore"),
            dimension_semantics=(pltpu.PARALLEL,),
        )(i_hbm, o_hbm)

    return kern(table, idx)


def _prep(x):
    return pl.pallas_call(
        _prep_body,
        out_shape=[
            jax.ShapeDtypeStruct((B, D), jnp.bfloat16),
            jax.ShapeDtypeStruct((D, B), jnp.bfloat16),
            jax.ShapeDtypeStruct((B, 1), jnp.float32),
            jax.ShapeDtypeStruct((1, B), jnp.float32),
        ],
    )(x)


def _similarity(xb, xt, n, nt):
    grid = (B // RBLK, B // CBLK)
    return pl.pallas_call(
        _sim_body,
        grid=grid,
        in_specs=[
            pl.BlockSpec((CBLK, D), lambda r, c: (c, 0)),
            pl.BlockSpec((D, RBLK), lambda r, c: (0, r)),
            pl.BlockSpec((CBLK, 1), lambda r, c: (c, 0)),
            pl.BlockSpec((1, RBLK), lambda r, c: (0, r)),
        ],
        out_specs=[
            pl.BlockSpec((RBLK, CBLK // GROUP, GROUP), lambda r, c: (r, c, 0)),
            pl.BlockSpec((1, RBLK), lambda r, c: (0, r)),
            pl.BlockSpec((RBLK, K), lambda r, c: (r, 0)),
        ],
        out_shape=[
            jax.ShapeDtypeStruct((B, NG, GROUP), jnp.float32),
            jax.ShapeDtypeStruct((1, B), jnp.float32),
            jax.ShapeDtypeStruct((B, K), jnp.int32),
        ],
        scratch_shapes=[
            pltpu.VMEM((1, RBLK), jnp.float32),
            pltpu.VMEM((B // CBLK, CBLK // GROUP, RBLK), jnp.float32),
        ],
        compiler_params=pltpu.CompilerParams(
            dimension_semantics=("parallel", "arbitrary")),
    )(xb, xt, n, nt)


def _select(cand, sidx, l):
    grid = (B // SEL_RBLK,)
    return pl.pallas_call(
        _select_body,
        grid=grid,
        in_specs=[
            pl.BlockSpec((SEL_RBLK, K * GROUP), lambda r: (r, 0)),
            pl.BlockSpec((SEL_RBLK, K), lambda r: (r, 0)),
            pl.BlockSpec((1, SEL_RBLK), lambda r: (0, r)),
        ],
        out_specs=[
            pl.BlockSpec((SEL_RBLK, K), lambda r: (r, 0)),
            pl.BlockSpec((SEL_RBLK, K), lambda r: (r, 0)),
        ],
        out_shape=[
            jax.ShapeDtypeStruct((B, K), jnp.float32),
            jax.ShapeDtypeStruct((B, K), jnp.int32),
        ],
        compiler_params=pltpu.CompilerParams(
            dimension_semantics=("parallel",)),
    )(cand, sidx, l)


def _combine(g, w):
    grid = (B // SEL_RBLK,)
    return pl.pallas_call(
        _combine_body,
        grid=grid,
        in_specs=[
            pl.BlockSpec((SEL_RBLK, K, GROUP), lambda r: (r, 0, 0)),
            pl.BlockSpec((SEL_RBLK, K), lambda r: (r, 0)),
        ],
        out_specs=pl.BlockSpec((SEL_RBLK, D), lambda r: (r, 0)),
        out_shape=jax.ShapeDtypeStruct((B, D), jnp.float32),
        compiler_params=pltpu.CompilerParams(
            dimension_semantics=("parallel",)),
    )(g, w)


@jax.jit
def kernel(sess_emb):
    xb, xt, n, nt = _prep(sess_emb)
    s, l, sidx = _similarity(xb, xt, n, nt)
    cand = _sc_gather(s, sidx.reshape(1, B * K),
                      value_dim=GROUP, window=128)
    cand = cand.reshape(B, K * GROUP)
    w, eidx = _select(cand, sidx, l)
    emb_pad = jnp.pad(sess_emb, ((0, 0), (0, GROUP - D)))
    g = _sc_gather(emb_pad, eidx.reshape(1, B * K), value_dim=GROUP,
                   window=128)
    return _combine(g.reshape(B, K, GROUP), w)


# speedup vs baseline: 1.0153x; 1.0153x over previous
import functools

import jax
import jax.numpy as jnp
from jax.experimental import pallas as pl
from jax.experimental.pallas import tpu as pltpu
from jax.experimental.pallas import tpu_sc as plsc

B = 16384
D = 32
K = 5
GROUP = 128
NG = B // GROUP
RBLK = 256
CBLK = 4096
SEL_RBLK = 512
NEG = -3.0e38


def _prep_body(x_ref, xb_ref, xt_ref, n_ref, nt_ref):
    x = x_ref[...]
    xb_ref[...] = x.astype(jnp.bfloat16)
    xt = x.T
    xt_ref[...] = xt.astype(jnp.bfloat16)
    n_ref[...] = jnp.sqrt(jnp.sum(x * x + 1e-6, axis=1, keepdims=True))
    nt_ref[...] = jnp.sqrt(jnp.sum(xt * xt + 1e-6, axis=0, keepdims=True))


def _sim_body(xkb_ref, xtq_ref, nk_ref, ntq_ref, s_ref, l_ref, sidx_ref,
              lsc, gsc):
    r = pl.program_id(0)
    c = pl.program_id(1)
    nc = pl.num_programs(1)
    gpc = CBLK // GROUP

    fz = jax.lax.dot_general(
        xkb_ref[...], xtq_ref[...],
        (((1,), (0,)), ((), ())),
        preferred_element_type=jnp.float32,
    )
    s = (fz * (1.0 / nk_ref[...])) * (1.0 / ntq_ref[...])
    s_ref[...] = s.T.reshape(RBLK, gpc, GROUP)

    g = jnp.max(s.reshape(gpc, GROUP, RBLK), axis=1)
    gsc[c] = g

    @pl.when(c == 0)
    def _():
        lsc[...] = jnp.zeros((1, RBLK), jnp.float32)

    lsc[...] += jnp.sum(jnp.exp(s), axis=0, keepdims=True)

    @pl.when(c == nc - 1)
    def _():
        l_ref[...] = lsc[...]
        cur = gsc[...]
        gid = (jax.lax.broadcasted_iota(jnp.int32, (nc, gpc, RBLK), 0) * gpc
               + jax.lax.broadcasted_iota(jnp.int32, (nc, gpc, RBLK), 1))
        picks = []
        for _j in range(K):
            mv = jnp.max(jnp.max(cur, axis=0), axis=0,
                         keepdims=True)
            am = jnp.min(jnp.min(jnp.where(cur >= mv, gid, NG), axis=0),
                         axis=0, keepdims=True)
            picks.append(am)
            cur = jnp.where(gid == am, NEG, cur)
        g5 = jnp.concatenate(picks, axis=0).T
        rows = r * RBLK + jax.lax.broadcasted_iota(jnp.int32, (RBLK, K), 0)
        sidx_ref[...] = rows * NG + g5


def _select_body(cand_ref, sidx_ref, l_ref, w_ref, ei_ref):
    r = pl.program_id(0)
    w = K * GROUP
    x = cand_ref[...]
    lane = jax.lax.broadcasted_iota(jnp.int32, (SEL_RBLK, w), 1)
    rows = r * SEL_RBLK + jax.lax.broadcasted_iota(
        jnp.int32, (SEL_RBLK, K), 0)
    g5 = sidx_ref[...] - rows * NG

    vals = []
    eidx = []
    for _t in range(K):
        mv = jnp.max(x, axis=-1, keepdims=True)
        am = jnp.min(jnp.where(x >= mv, lane, w), axis=-1,
                     keepdims=True)
        slot = am // GROUP
        off = am - slot * GROUP
        gsel = jnp.zeros_like(slot)
        for j in range(K):
            gsel = jnp.where(slot == j, g5[:, j:j + 1], gsel)
        eidx.append(gsel * GROUP + off)
        vals.append(mv)
        x = jnp.where(lane == am, NEG, x)

    v = jnp.concatenate(vals, axis=1)
    lcol = l_ref[...].T
    p = jnp.exp(v) / lcol
    pm = jnp.max(p, axis=-1, keepdims=True)
    e = jnp.exp(p - pm)
    w_ref[...] = e / jnp.sum(e, axis=-1, keepdims=True)
    ei_ref[...] = jnp.concatenate(eidx, axis=1)


def _combine_body(g_ref, w_ref, o_ref):
    g = g_ref[...]
    w = w_ref[...]
    acc = w[:, 0:1] * g[:, 0:D]
    for j in range(1, K):
        acc = acc + w[:, j:j + 1] * g[:, j * GROUP:j * GROUP + D]
    o_ref[...] = acc


def _sc_gather(table, idx, value_dim, window):
    n = idx.shape[1]
    rows = table.size // value_dim
    mesh = plsc.VectorSubcoreMesh(core_axis_name="core",
                                  subcore_axis_name="subcore")

    @pl.kernel(out_type=jax.ShapeDtypeStruct((n, value_dim), table.dtype),
               mesh=mesh)
    def kern(x_hbm, i_hbm, o_hbm):
        x2d = x_hbm.reshape(rows, value_dim)

        def body(i_vmem, o_vmem):
            pltpu.sync_copy(x2d.at[i_vmem.at[0]], o_vmem)

        pltpu.emit_pipeline(
            body,
            grid=(n // window,),
            in_specs=[pl.BlockSpec((1, window), index_map=lambda i: (0, i))],
            out_specs=[pl.BlockSpec((window, value_dim),
                                    index_map=lambda i: (i, 0))],
            core_axis_name=("core", "subcore"),
            dimension_semantics=(pltpu.PARALLEL,),
        )(i_hbm, o_hbm)

    return kern(table, idx)


def _prep(x):
    return pl.pallas_call(
        _prep_body,
        out_shape=[
            jax.ShapeDtypeStruct((B, D), jnp.bfloat16),
            jax.ShapeDtypeStruct((D, B), jnp.bfloat16),
            jax.ShapeDtypeStruct((B, 1), jnp.float32),
            jax.ShapeDtypeStruct((1, B), jnp.float32),
        ],
    )(x)


def _similarity(xb, xt, n, nt):
    grid = (B // RBLK, B // CBLK)
    return pl.pallas_call(
        _sim_body,
        grid=grid,
        in_specs=[
            pl.BlockSpec((CBLK, D), lambda r, c: (c, 0)),
            pl.BlockSpec((D, RBLK), lambda r, c: (0, r)),
            pl.BlockSpec((CBLK, 1), lambda r, c: (c, 0)),
            pl.BlockSpec((1, RBLK), lambda r, c: (0, r)),
        ],
        out_specs=[
            pl.BlockSpec((RBLK, CBLK // GROUP, GROUP), lambda r, c: (r, c, 0)),
            pl.BlockSpec((1, RBLK), lambda r, c: (0, r)),
            pl.BlockSpec((RBLK, K), lambda r, c: (r, 0)),
        ],
        out_shape=[
            jax.ShapeDtypeStruct((B, NG, GROUP), jnp.float32),
            jax.ShapeDtypeStruct((1, B), jnp.float32),
            jax.ShapeDtypeStruct((B, K), jnp.int32),
        ],
        scratch_shapes=[
            pltpu.VMEM((1, RBLK), jnp.float32),
            pltpu.VMEM((B // CBLK, CBLK // GROUP, RBLK), jnp.float32),
        ],
        compiler_params=pltpu.CompilerParams(
            dimension_semantics=("parallel", "arbitrary")),
    )(xb, xt, n, nt)


def _select(cand, sidx, l):
    grid = (B // SEL_RBLK,)
    return pl.pallas_call(
        _select_body,
        grid=grid,
        in_specs=[
            pl.BlockSpec((SEL_RBLK, K * GROUP), lambda r: (r, 0)),
            pl.BlockSpec((SEL_RBLK, K), lambda r: (r, 0)),
            pl.BlockSpec((1, SEL_RBLK), lambda r: (0, r)),
        ],
        out_specs=[
            pl.BlockSpec((SEL_RBLK, K), lambda r: (r, 0)),
            pl.BlockSpec((SEL_RBLK, K), lambda r: (r, 0)),
        ],
        out_shape=[
            jax.ShapeDtypeStruct((B, K), jnp.float32),
            jax.ShapeDtypeStruct((B, K), jnp.int32),
        ],
        compiler_params=pltpu.CompilerParams(
            dimension_semantics=("parallel",)),
    )(cand, sidx, l)


def _combine(g, w):
    grid = (B // SEL_RBLK,)
    return pl.pallas_call(
        _combine_body,
        grid=grid,
        in_specs=[
            pl.BlockSpec((SEL_RBLK, K * GROUP), lambda r: (r, 0)),
            pl.BlockSpec((SEL_RBLK, K), lambda r: (r, 0)),
        ],
        out_specs=pl.BlockSpec((SEL_RBLK, D), lambda r: (r, 0)),
        out_shape=jax.ShapeDtypeStruct((B, D), jnp.float32),
        compiler_params=pltpu.CompilerParams(
            dimension_semantics=("parallel",)),
    )(g, w)


@jax.jit
def kernel(sess_emb):
    xb, xt, n, nt = _prep(sess_emb)
    s, l, sidx = _similarity(xb, xt, n, nt)
    cand = _sc_gather(s, sidx.reshape(1, B * K),
                      value_dim=GROUP, window=128)
    cand = cand.reshape(B, K * GROUP)
    w, eidx = _select(cand, sidx, l)
    emb_pad = jnp.pad(sess_emb, ((0, 0), (0, GROUP - D)))
    g = _sc_gather(emb_pad, eidx.reshape(1, B * K), value_dim=GROUP,
                   window=128)
    return _combine(g.reshape(B, K * GROUP), w)
